# baseline (speedup 1.0000x reference)
import functools

import jax
import jax.numpy as jnp
from jax import lax
from jax.experimental import pallas as pl
from jax.experimental.pallas import tpu as pltpu
from jax.experimental.pallas import tpu_sc as plsc

N_NODES = 100000
H = 128

NC = 2
NS = 16
NW = NC * NS
CHUNK = 128
CHUNKS_PER_W = 25
ROWS_PER_W = CHUNK * CHUNKS_PER_W
N_PAD = NW * ROWS_PER_W


def _sc_gather_body(h_hbm, c_hbm, i0_hbm, i1_hbm,
                    oh0, oh1, oc0, oc1,
                    i0v, i1v, bh0, bh1, bc0, bc1,
                    semg, semsh, semsc):
    cid = lax.axis_index("c")
    sid = lax.axis_index("s")
    wid = sid * NC + cid
    pltpu.sync_copy(i0_hbm.at[wid], i0v)
    pltpu.sync_copy(i1_hbm.at[wid], i1v)

    def drain_h():
        pltpu.make_async_copy(bh0, oh0.at[pl.ds(0, CHUNK)], semsh).wait()
        pltpu.make_async_copy(bh1, oh1.at[pl.ds(0, CHUNK)], semsh).wait()

    def drain_c():
        pltpu.make_async_copy(bc0, oc0.at[pl.ds(0, CHUNK)], semsc).wait()
        pltpu.make_async_copy(bc1, oc1.at[pl.ds(0, CHUNK)], semsc).wait()

    def chunk(j, carry):
        base = wid * ROWS_PER_W + j * CHUNK
        gh0 = pltpu.async_copy(h_hbm.at[i0v.at[j]], bh0, semg)
        gh1 = pltpu.async_copy(h_hbm.at[i1v.at[j]], bh1, semg)
        gh0.wait()
        gh1.wait()
        pltpu.async_copy(bh0, oh0.at[pl.ds(base, CHUNK)], semsh)
        pltpu.async_copy(bh1, oh1.at[pl.ds(base, CHUNK)], semsh)
        pl.when(j > 0)(drain_c)
        gc0 = pltpu.async_copy(c_hbm.at[i0v.at[j]], bc0, semg)
        gc1 = pltpu.async_copy(c_hbm.at[i1v.at[j]], bc1, semg)
        gc0.wait()
        gc1.wait()
        pltpu.async_copy(bc0, oc0.at[pl.ds(base, CHUNK)], semsc)
        pltpu.async_copy(bc1, oc1.at[pl.ds(base, CHUNK)], semsc)
        drain_h()
        return carry

    lax.fori_loop(0, CHUNKS_PER_W, chunk, 0)
    drain_c()


@jax.jit
def _sc_gather(h, c, idx0, idx1):
    mesh = plsc.VectorSubcoreMesh(core_axis_name="c", subcore_axis_name="s")
    row = jax.ShapeDtypeStruct((N_PAD, H), jnp.float32)
    fn = pl.kernel(
        _sc_gather_body,
        mesh=mesh,
        out_type=(row, row, row, row),
        scratch_types=[
            pltpu.VMEM((CHUNKS_PER_W, CHUNK), jnp.int32),
            pltpu.VMEM((CHUNKS_PER_W, CHUNK), jnp.int32),
            pltpu.VMEM((CHUNK, H), jnp.float32),
            pltpu.VMEM((CHUNK, H), jnp.float32),
            pltpu.VMEM((CHUNK, H), jnp.float32),
            pltpu.VMEM((CHUNK, H), jnp.float32),
            pltpu.SemaphoreType.DMA,
            pltpu.SemaphoreType.DMA,
            pltpu.SemaphoreType.DMA,
        ],
    )
    return fn(h, c, idx0, idx1)


BLK = 1024


def _dense_body(x_ref, h0_ref, h1_ref, c0_ref, c1_ref,
                w_ref, u0_ref, u1_ref, b_ref, f0_ref, f1_ref, bf_ref,
                hout_ref, cout_ref):
    x = x_ref[...]
    h0 = h0_ref[...]
    h1 = h1_ref[...]
    c0 = c0_ref[...]
    c1 = c1_ref[...]
    f32 = jnp.float32
    iou = (jnp.dot(x, w_ref[...], preferred_element_type=f32)
           + jnp.dot(h0, u0_ref[...], preferred_element_type=f32)
           + jnp.dot(h1, u1_ref[...], preferred_element_type=f32)
           + b_ref[...])
    fpre = (jnp.dot(h0, f0_ref[...], preferred_element_type=f32)
            + jnp.dot(h1, f1_ref[...], preferred_element_type=f32)
            + bf_ref[...])
    f = jax.nn.sigmoid(fpre)
    c_agg = f[:, :H] * c0 + f[:, H:] * c1
    i = jax.nn.sigmoid(iou[:, :H])
    o = jax.nn.sigmoid(iou[:, H:2 * H])
    u = jnp.tanh(iou[:, 2 * H:])
    c_new = i * u + c_agg
    hout_ref[...] = o * jnp.tanh(c_new)
    cout_ref[...] = c_new


@jax.jit
def _dense(x, h0, h1, c0, c1, W_iou, Um0, Um1, b_iou, Uf0, Uf1, bf):
    n = x.shape[0]
    grid = (pl.cdiv(n, BLK),)
    row_spec = pl.BlockSpec((BLK, H), lambda i: (i, 0))
    full = lambda s: pl.BlockSpec(s, lambda i: (0, 0))
    return pl.pallas_call(
        _dense_body,
        grid=grid,
        in_specs=[
            row_spec, row_spec, row_spec, row_spec, row_spec,
            full((H, 3 * H)), full((H, 3 * H)), full((H, 3 * H)),
            full((1, 3 * H)),
            full((H, 2 * H)), full((H, 2 * H)), full((1, 2 * H)),
        ],
        out_specs=[
            pl.BlockSpec((BLK, H), lambda i: (i, 0)),
            pl.BlockSpec((BLK, H), lambda i: (i, 0)),
        ],
        out_shape=[
            jax.ShapeDtypeStruct((n, H), jnp.float32),
            jax.ShapeDtypeStruct((n, H), jnp.float32),
        ],
        compiler_params=pltpu.CompilerParams(
            dimension_semantics=("arbitrary",),
        ),
    )(x, h0, h1, c0, c1, W_iou, Um0, Um1, b_iou, Uf0, Uf1, bf)


def kernel(x, h, c, child_idx, W_iou, Um0_iou, Um1_iou, b_iou, U_f_w, U_f_b):
    idx = child_idx.astype(jnp.int32)
    pad = N_PAD - N_NODES
    idx0 = jnp.pad(idx[:, 0], (0, pad)).reshape(NW, CHUNKS_PER_W, CHUNK)
    idx1 = jnp.pad(idx[:, 1], (0, pad)).reshape(NW, CHUNKS_PER_W, CHUNK)
    h0, h1, c0, c1 = _sc_gather(h, c, idx0, idx1)
    Uf0 = U_f_w[:H, :]
    Uf1 = U_f_w[H:, :]
    bf = U_f_b.reshape(1, 2 * H)
    h_new, c_new = _dense(x, h0, h1, c0, c1,
                          W_iou, Um0_iou, Um1_iou, b_iou, Uf0, Uf1, bf)
    return h_new, c_new


# speedup vs baseline: 1.3401x; 1.3401x over previous
import functools

import jax
import jax.numpy as jnp
from jax import lax
from jax.experimental import pallas as pl
from jax.experimental.pallas import tpu as pltpu
from jax.experimental.pallas import tpu_sc as plsc

N_NODES = 100000
H = 128

NC = 2
NS = 16
NW = NC * NS
CHUNK = 64
CHUNKS_PER_W = 50
ROWS_PER_W = CHUNK * CHUNKS_PER_W
N_PAD = NW * ROWS_PER_W
PAIRS = CHUNKS_PER_W // 2


def _sc_gather_body(h_hbm, c_hbm, i0_hbm, i1_hbm,
                    oh0, oh1, oc0, oc1,
                    i0v, i1v, bufs, semga, semgb, semsa, semsb):
    cid = lax.axis_index("c")
    sid = lax.axis_index("s")
    wid = sid * NC + cid
    pltpu.sync_copy(i0_hbm.at[wid], i0v)
    pltpu.sync_copy(i1_hbm.at[wid], i1v)

    outs = (oh0, oh1, oc0, oc1)

    def drain_stores(s, sem):
        for t in range(4):
            pltpu.make_async_copy(
                bufs.at[s, t], outs[t].at[pl.ds(0, CHUNK)], sem).wait()

    def fire_gathers(s, j, sem):
        return [
            pltpu.async_copy(h_hbm.at[i0v.at[j]], bufs.at[s, 0], sem),
            pltpu.async_copy(h_hbm.at[i1v.at[j]], bufs.at[s, 1], sem),
            pltpu.async_copy(c_hbm.at[i0v.at[j]], bufs.at[s, 2], sem),
            pltpu.async_copy(c_hbm.at[i1v.at[j]], bufs.at[s, 3], sem),
        ]

    def fire_stores(s, j, sem):
        base = wid * ROWS_PER_W + j * CHUNK
        for t in range(4):
            pltpu.async_copy(bufs.at[s, t], outs[t].at[pl.ds(base, CHUNK)],
                             sem)

    def pair(p, carry):
        ja = 2 * p
        jb = 2 * p + 1
        pl.when(p > 0)(lambda: drain_stores(0, semsa))
        ga = fire_gathers(0, ja, semga)
        pl.when(p > 0)(lambda: drain_stores(1, semsb))
        gb = fire_gathers(1, jb, semgb)
        for g in ga:
            g.wait()
        fire_stores(0, ja, semsa)
        for g in gb:
            g.wait()
        fire_stores(1, jb, semsb)
        return carry

    lax.fori_loop(0, PAIRS, pair, 0)
    drain_stores(0, semsa)
    drain_stores(1, semsb)


@jax.jit
def _sc_gather(h, c, idx0, idx1):
    mesh = plsc.VectorSubcoreMesh(core_axis_name="c", subcore_axis_name="s")
    row = jax.ShapeDtypeStruct((N_PAD, H), jnp.float32)
    fn = pl.kernel(
        _sc_gather_body,
        mesh=mesh,
        out_type=(row, row, row, row),
        scratch_types=[
            pltpu.VMEM((CHUNKS_PER_W, CHUNK), jnp.int32),
            pltpu.VMEM((CHUNKS_PER_W, CHUNK), jnp.int32),
            pltpu.VMEM((2, 4, CHUNK, H), jnp.float32),
            pltpu.SemaphoreType.DMA,
            pltpu.SemaphoreType.DMA,
            pltpu.SemaphoreType.DMA,
            pltpu.SemaphoreType.DMA,
        ],
    )
    return fn(h, c, idx0, idx1)


BLK = 1024


def _dense_body(x_ref, h0_ref, h1_ref, c0_ref, c1_ref,
                w_ref, u0_ref, u1_ref, b_ref, f0_ref, f1_ref, bf_ref,
                hout_ref, cout_ref):
    x = x_ref[...]
    h0 = h0_ref[...]
    h1 = h1_ref[...]
    c0 = c0_ref[...]
    c1 = c1_ref[...]
    f32 = jnp.float32
    iou = (jnp.dot(x, w_ref[...], preferred_element_type=f32)
           + jnp.dot(h0, u0_ref[...], preferred_element_type=f32)
           + jnp.dot(h1, u1_ref[...], preferred_element_type=f32)
           + b_ref[...])
    fpre = (jnp.dot(h0, f0_ref[...], preferred_element_type=f32)
            + jnp.dot(h1, f1_ref[...], preferred_element_type=f32)
            + bf_ref[...])
    f = jax.nn.sigmoid(fpre)
    c_agg = f[:, :H] * c0 + f[:, H:] * c1
    i = jax.nn.sigmoid(iou[:, :H])
    o = jax.nn.sigmoid(iou[:, H:2 * H])
    u = jnp.tanh(iou[:, 2 * H:])
    c_new = i * u + c_agg
    hout_ref[...] = o * jnp.tanh(c_new)
    cout_ref[...] = c_new


@jax.jit
def _dense(x, h0, h1, c0, c1, W_iou, Um0, Um1, b_iou, Uf0, Uf1, bf):
    n = x.shape[0]
    grid = (pl.cdiv(n, BLK),)
    row_spec = pl.BlockSpec((BLK, H), lambda i: (i, 0))
    full = lambda s: pl.BlockSpec(s, lambda i: (0, 0))
    return pl.pallas_call(
        _dense_body,
        grid=grid,
        in_specs=[
            row_spec, row_spec, row_spec, row_spec, row_spec,
            full((H, 3 * H)), full((H, 3 * H)), full((H, 3 * H)),
            full((1, 3 * H)),
            full((H, 2 * H)), full((H, 2 * H)), full((1, 2 * H)),
        ],
        out_specs=[
            pl.BlockSpec((BLK, H), lambda i: (i, 0)),
            pl.BlockSpec((BLK, H), lambda i: (i, 0)),
        ],
        out_shape=[
            jax.ShapeDtypeStruct((n, H), jnp.float32),
            jax.ShapeDtypeStruct((n, H), jnp.float32),
        ],
        compiler_params=pltpu.CompilerParams(
            dimension_semantics=("arbitrary",),
        ),
    )(x, h0, h1, c0, c1, W_iou, Um0, Um1, b_iou, Uf0, Uf1, bf)


def kernel(x, h, c, child_idx, W_iou, Um0_iou, Um1_iou, b_iou, U_f_w, U_f_b):
    idx = child_idx.astype(jnp.int32)
    pad = N_PAD - N_NODES
    idx0 = jnp.pad(idx[:, 0], (0, pad)).reshape(NW, CHUNKS_PER_W, CHUNK)
    idx1 = jnp.pad(idx[:, 1], (0, pad)).reshape(NW, CHUNKS_PER_W, CHUNK)
    h0, h1, c0, c1 = _sc_gather(h, c, idx0, idx1)
    Uf0 = U_f_w[:H, :]
    Uf1 = U_f_w[H:, :]
    bf = U_f_b.reshape(1, 2 * H)
    h_new, c_new = _dense(x, h0, h1, c0, c1,
                          W_iou, Um0_iou, Um1_iou, b_iou, Uf0, Uf1, bf)
    return h_new, c_new
